# baseline (speedup 1.0000x reference)
import functools

import jax
import jax.numpy as jnp
from jax import lax
from jax.experimental import pallas as pl
from jax.experimental.pallas import tpu as pltpu
from jax.experimental.pallas import tpu_sc as plsc

N = 10000
E = 320000
D = 128
R = 8

CH = 16
NCH = D // CH
ROWS = R * N
TRASH = ROWS
SROWS = ROWS + 16
NC = 2
NS = 16
PASSES = NCH // NC

E2 = 2 * E
K = 512
TPE = 40960
NBLK = TPE // K
E2P = NS * TPE
NG = K // 16
SUB = K // 128
STRIPE = ROWS // NS


_SC_PARAMS = pltpu.CompilerParams(
    needs_layout_passes=False, use_tc_tiling_on_sc=False
)
_MESH = dict(core_axis_name="c", subcore_axis_name="s",
             num_cores=NC, num_subcores=NS)

TPD = E2P // (NC * NS)
NBLKD = TPD // K
NBT = E2P // K


def _sc_degrees(epack, z2):
    @functools.partial(
        pl.kernel,
        mesh=plsc.VectorSubcoreMesh(**_MESH),
        compiler_params=_SC_PARAMS,
        out_type=jax.ShapeDtypeStruct((NC * NS, ROWS), jnp.float32),
        scratch_types=[
            pltpu.VMEM((2, 3, K), jnp.int32),
            pltpu.VMEM((SROWS,), jnp.float32),
            pltpu.SemaphoreType.DMA,
        ],
    )
    def deg_kernel(ep_h, z2_h, deg_out, est, hist, sem_e):
        cid = lax.axis_index("c")
        tid = lax.axis_index("s")
        wid = cid * NS + tid
        bbase = wid * NBLKD
        pltpu.sync_copy(z2_h, hist)
        ones = jnp.ones((16,), jnp.float32)
        pltpu.async_copy(ep_h.at[bbase], est.at[0], sem_e)

        @pl.loop(0, NBLKD)
        def _(b):
            q = lax.rem(b, 2)
            pltpu.make_async_copy(ep_h.at[0], est.at[0], sem_e).wait()
            nxt = jnp.minimum(b + 1, NBLKD - 1)
            pltpu.async_copy(ep_h.at[bbase + nxt], est.at[1 - q], sem_e)
            for j in range(NG):
                d = est[q, 1, pl.ds(j * 16, 16)]
                t = est[q, 2, pl.ds(j * 16, 16)]
                plsc.addupdate_scatter(hist, [t * N + d], ones)

        pltpu.make_async_copy(ep_h.at[0], est.at[0], sem_e).wait()
        pltpu.sync_copy(hist.at[pl.ds(0, ROWS)], deg_out.at[wid])

    return deg_kernel(epack, z2)


def _sc_segment_sums(xflat, epack, z1):
    @functools.partial(
        pl.kernel,
        mesh=plsc.VectorSubcoreMesh(**_MESH),
        compiler_params=_SC_PARAMS,
        out_type=jax.ShapeDtypeStruct((ROWS, NCH, CH), jnp.float32),
        scratch_types=[
            pltpu.VMEM((2, 3, K), jnp.int32),
            pltpu.VMEM((2, K), jnp.int32),
            pltpu.VMEM((2, K), jnp.int32),
            pltpu.VMEM((2, K, CH), jnp.float32),
            pltpu.VMEM_SHARED((SROWS, CH), jnp.float32),
            pltpu.SemaphoreType.DMA,
            pltpu.SemaphoreType.DMA,
            pltpu.SemaphoreType.DMA,
        ],
    )
    def sc_kernel(xf, ep_h, z1_h, s_out,
                  est, idx3, gidx3, rows2, s_sh, sem_e, sem_g, sem_s):
        cid = lax.axis_index("c")
        tid = lax.axis_index("s")
        bbase = tid * NBLK

        def start_edges(b, q):
            pltpu.async_copy(ep_h.at[bbase + b], est.at[q], sem_e)

        def wait_edges():
            pltpu.make_async_copy(ep_h.at[0], est.at[0], sem_e).wait()

        def compute_indices(q, gbase):
            for j in range(NG):
                s = est[q, 0, pl.ds(j * 16, 16)]
                d = est[q, 1, pl.ds(j * 16, 16)]
                t = est[q, 2, pl.ds(j * 16, 16)]
                idx3[q, pl.ds(j * 16, 16)] = t * N + d
                gidx3[q, pl.ds(j * 16, 16)] = jnp.where(s >= N, TRASH, s + gbase)

        def fire_gathers(q):
            pltpu.async_copy(xf.at[gidx3.at[q]], rows2.at[q], sem_g)

        def wait_gathers(q):
            pltpu.make_async_copy(xf.at[pl.ds(0, K)], rows2.at[q], sem_g).wait()

        def fire_scatters(q):
            pltpu.async_copy(rows2.at[q], s_sh.at[idx3.at[q]], sem_s, add=True)

        def wait_scatters(q):
            pltpu.make_async_copy(xf.at[pl.ds(0, K)], rows2.at[q], sem_s).wait()

        for p in range(PASSES):
            chunk = cid * PASSES + p
            gbase = chunk * N

            pltpu.sync_copy(z1_h, s_sh.at[pl.ds(tid * STRIPE, STRIPE)])
            plsc.subcore_barrier()

            start_edges(0, 0)
            wait_edges()
            compute_indices(0, gbase)
            start_edges(1, 1)
            fire_gathers(0)
            wait_edges()
            compute_indices(1, gbase)
            start_edges(2, 0)
            wait_gathers(0)
            fire_gathers(1)
            fire_scatters(0)

            @pl.loop(1, NBLK - 2)
            def _(b):
                q = lax.rem(b, 2)
                qn = 1 - q
                wait_scatters(qn)
                wait_edges()
                compute_indices(qn, gbase)
                start_edges(b + 2, q)
                wait_gathers(q)
                fire_gathers(qn)
                fire_scatters(q)

            wait_scatters(1)
            wait_edges()
            compute_indices(1, gbase)
            wait_gathers(0)
            fire_gathers(1)
            fire_scatters(0)
            wait_scatters(0)
            wait_gathers(1)
            fire_scatters(1)
            wait_scatters(1)

            plsc.subcore_barrier()
            pltpu.sync_copy(
                s_sh.at[pl.ds(tid * STRIPE, STRIPE)],
                s_out.at[pl.ds(tid * STRIPE, STRIPE), chunk],
            )

    return sc_kernel(xflat, epack, z1)


BN = 1000


def _tc_body(s_ref, deg_ref, x_ref, w_ref, sw_ref, o_ref):
    deg = jnp.sum(deg_ref[...], axis=2)
    dinv = jnp.where(deg > 0.0, 1.0 / jnp.maximum(deg, 1.0), 0.0)
    acc = jnp.dot(x_ref[...], sw_ref[...], preferred_element_type=jnp.float32)
    for r in range(R):
        acc = acc + jnp.dot(
            s_ref[r] * dinv[:, r : r + 1], w_ref[r],
            preferred_element_type=jnp.float32,
        )
    o_ref[...] = acc


def kernel(x, edge_index, edge_type, weight, self_weight):
    src0 = edge_index[0]
    dst0 = edge_index[1]
    npad = E2P - E2
    srcp = jnp.concatenate([src0, dst0, jnp.full((npad,), N, jnp.int32)])
    dstp = jnp.concatenate([dst0, src0, jnp.zeros((npad,), jnp.int32)])
    etp = jnp.concatenate([edge_type, edge_type, jnp.full((npad,), R, jnp.int32)])
    epack = (
        jnp.stack([srcp, dstp, etp]).reshape(3, NBT, K).transpose(1, 0, 2)
    )
    xT = x.reshape(N, NCH, CH).transpose(1, 0, 2).reshape(NCH * N, CH)
    xflat = jnp.concatenate([xT, jnp.zeros((CH, CH), jnp.float32)])
    z1 = jnp.zeros((STRIPE, CH), jnp.float32)
    z2 = jnp.zeros((SROWS,), jnp.float32)

    deg32 = _sc_degrees(epack, z2)
    s_rows = _sc_segment_sums(xflat, epack, z1)
    s3 = s_rows.reshape(R, N, D)
    deg3 = deg32.reshape(NC * NS, R, N).transpose(2, 1, 0)

    out = pl.pallas_call(
        _tc_body,
        grid=(N // BN,),
        in_specs=[
            pl.BlockSpec((R, BN, D), lambda i: (0, i, 0)),
            pl.BlockSpec((BN, R, NC * NS), lambda i: (i, 0, 0)),
            pl.BlockSpec((BN, D), lambda i: (i, 0)),
            pl.BlockSpec((R, D, D), lambda i: (0, 0, 0)),
            pl.BlockSpec((D, D), lambda i: (0, 0)),
        ],
        out_specs=pl.BlockSpec((BN, D), lambda i: (i, 0)),
        out_shape=jax.ShapeDtypeStruct((N, D), jnp.float32),
        compiler_params=pltpu.CompilerParams(
            dimension_semantics=("arbitrary",)
        ),
    )(s3, deg3, x, weight, self_weight)
    return out


# speedup vs baseline: 1.6340x; 1.6340x over previous
import functools

import jax
import jax.numpy as jnp
from jax import lax
from jax.experimental import pallas as pl
from jax.experimental.pallas import tpu as pltpu
from jax.experimental.pallas import tpu_sc as plsc

N = 10000
E = 320000
D = 128
R = 8

CH = 32
NCH = D // CH
ADT = jnp.bfloat16
ROWS = R * N
TRASH = ROWS
GTRASH = NCH * N
SROWS = ROWS + 16
NC = 2
NS = 16
PASSES = NCH // NC

E2 = 2 * E
K = 512
TPE = 40960
NBLK = TPE // K
E2P = NS * TPE
NG = K // 16
SUB = K // 128
STRIPE = ROWS // NS


_SC_PARAMS = pltpu.CompilerParams(
    needs_layout_passes=False, use_tc_tiling_on_sc=False
)
_MESH = dict(core_axis_name="c", subcore_axis_name="s",
             num_cores=NC, num_subcores=NS)

TPD = E2P // (NC * NS)
NBLKD = TPD // K
NBT = E2P // K


def _sc_degrees(epack, z2):
    @functools.partial(
        pl.kernel,
        mesh=plsc.VectorSubcoreMesh(**_MESH),
        compiler_params=_SC_PARAMS,
        out_type=jax.ShapeDtypeStruct((NC * NS, ROWS), jnp.float32),
        scratch_types=[
            pltpu.VMEM((2, 3, K), jnp.int32),
            pltpu.VMEM((SROWS,), jnp.float32),
            pltpu.SemaphoreType.DMA,
        ],
    )
    def deg_kernel(ep_h, z2_h, deg_out, est, hist, sem_e):
        cid = lax.axis_index("c")
        tid = lax.axis_index("s")
        wid = cid * NS + tid
        bbase = wid * NBLKD
        pltpu.sync_copy(z2_h, hist)
        ones = jnp.ones((16,), jnp.float32)
        pltpu.async_copy(ep_h.at[bbase], est.at[0], sem_e)

        @pl.loop(0, NBLKD)
        def _(b):
            q = lax.rem(b, 2)
            pltpu.make_async_copy(ep_h.at[0], est.at[0], sem_e).wait()
            nxt = jnp.minimum(b + 1, NBLKD - 1)
            pltpu.async_copy(ep_h.at[bbase + nxt], est.at[1 - q], sem_e)
            for j in range(NG):
                d = est[q, 1, pl.ds(j * 16, 16)]
                t = est[q, 2, pl.ds(j * 16, 16)]
                plsc.addupdate_scatter(hist, [t * N + d], ones)

        pltpu.make_async_copy(ep_h.at[0], est.at[0], sem_e).wait()
        pltpu.sync_copy(hist.at[pl.ds(0, ROWS)], deg_out.at[wid])

    return deg_kernel(epack, z2)


def _sc_segment_sums(xflat, epack, z1):
    @functools.partial(
        pl.kernel,
        mesh=plsc.VectorSubcoreMesh(**_MESH),
        compiler_params=_SC_PARAMS,
        out_type=jax.ShapeDtypeStruct((ROWS, NCH, CH), ADT),
        scratch_types=[
            pltpu.VMEM((2, 3, K), jnp.int32),
            pltpu.VMEM((2, K), jnp.int32),
            pltpu.VMEM((2, K), jnp.int32),
            pltpu.VMEM((2, K, CH), ADT),
            pltpu.VMEM_SHARED((SROWS, CH), ADT),
            pltpu.SemaphoreType.DMA,
            pltpu.SemaphoreType.DMA,
            pltpu.SemaphoreType.DMA,
        ],
    )
    def sc_kernel(xf, ep_h, z1_h, s_out,
                  est, idx3, gidx3, rows2, s_sh, sem_e, sem_g, sem_s):
        cid = lax.axis_index("c")
        tid = lax.axis_index("s")
        bbase = tid * NBLK

        def start_edges(b, q):
            pltpu.async_copy(ep_h.at[bbase + b], est.at[q], sem_e)

        def wait_edges():
            pltpu.make_async_copy(ep_h.at[0], est.at[0], sem_e).wait()

        def compute_indices(q, gbase):
            for j in range(NG):
                s = est[q, 0, pl.ds(j * 16, 16)]
                d = est[q, 1, pl.ds(j * 16, 16)]
                t = est[q, 2, pl.ds(j * 16, 16)]
                idx3[q, pl.ds(j * 16, 16)] = t * N + d
                gidx3[q, pl.ds(j * 16, 16)] = jnp.where(s >= N, GTRASH, s + gbase)

        def fire_gathers(q):
            pltpu.async_copy(xf.at[gidx3.at[q]], rows2.at[q], sem_g)

        def wait_gathers(q):
            pltpu.make_async_copy(xf.at[pl.ds(0, K)], rows2.at[q], sem_g).wait()

        def fire_scatters(q):
            pltpu.async_copy(rows2.at[q], s_sh.at[idx3.at[q]], sem_s, add=True)

        def wait_scatters(q):
            pltpu.make_async_copy(xf.at[pl.ds(0, K)], rows2.at[q], sem_s).wait()

        for p in range(PASSES):
            chunk = cid * PASSES + p
            gbase = chunk * N

            pltpu.sync_copy(z1_h, s_sh.at[pl.ds(tid * STRIPE, STRIPE)])
            plsc.subcore_barrier()

            start_edges(0, 0)
            wait_edges()
            compute_indices(0, gbase)
            start_edges(1, 1)
            fire_gathers(0)
            wait_edges()
            compute_indices(1, gbase)
            start_edges(2, 0)
            wait_gathers(0)
            fire_gathers(1)
            fire_scatters(0)

            @pl.loop(1, NBLK - 2)
            def _(b):
                q = lax.rem(b, 2)
                qn = 1 - q
                wait_scatters(qn)
                wait_edges()
                compute_indices(qn, gbase)
                start_edges(b + 2, q)
                wait_gathers(q)
                fire_gathers(qn)
                fire_scatters(q)

            wait_scatters(1)
            wait_edges()
            compute_indices(1, gbase)
            wait_gathers(0)
            fire_gathers(1)
            fire_scatters(0)
            wait_scatters(0)
            wait_gathers(1)
            fire_scatters(1)
            wait_scatters(1)

            plsc.subcore_barrier()
            pltpu.sync_copy(
                s_sh.at[pl.ds(tid * STRIPE, STRIPE)],
                s_out.at[pl.ds(tid * STRIPE, STRIPE), chunk],
            )

    return sc_kernel(xflat, epack, z1)


BN = 1000


def _tc_body(s_ref, deg_ref, x_ref, w_ref, sw_ref, o_ref):
    deg = jnp.sum(deg_ref[...], axis=2)
    dinv = jnp.where(deg > 0.0, 1.0 / jnp.maximum(deg, 1.0), 0.0)
    acc = jnp.dot(x_ref[...], sw_ref[...], preferred_element_type=jnp.float32)
    for r in range(R):
        acc = acc + jnp.dot(
            s_ref[r].astype(jnp.float32) * dinv[:, r : r + 1], w_ref[r],
            preferred_element_type=jnp.float32,
        )
    o_ref[...] = acc


def kernel(x, edge_index, edge_type, weight, self_weight):
    src0 = edge_index[0]
    dst0 = edge_index[1]
    npad = E2P - E2
    srcp = jnp.concatenate([src0, dst0, jnp.full((npad,), N, jnp.int32)])
    dstp = jnp.concatenate([dst0, src0, jnp.zeros((npad,), jnp.int32)])
    etp = jnp.concatenate([edge_type, edge_type, jnp.full((npad,), R, jnp.int32)])
    epack = (
        jnp.stack([srcp, dstp, etp]).reshape(3, NBT, K).transpose(1, 0, 2)
    )
    xT = (
        x.astype(ADT).reshape(N, NCH, CH).transpose(1, 0, 2).reshape(NCH * N, CH)
    )
    xflat = jnp.concatenate([xT, jnp.zeros((16, CH), ADT)])
    z1 = jnp.zeros((STRIPE, CH), ADT)
    z2 = jnp.zeros((SROWS,), jnp.float32)

    deg32 = _sc_degrees(epack, z2)
    s_rows = _sc_segment_sums(xflat, epack, z1)
    s3 = s_rows.reshape(R, N, D)
    deg3 = deg32.reshape(NC * NS, R, N).transpose(2, 1, 0)

    out = pl.pallas_call(
        _tc_body,
        grid=(N // BN,),
        in_specs=[
            pl.BlockSpec((R, BN, D), lambda i: (0, i, 0)),
            pl.BlockSpec((BN, R, NC * NS), lambda i: (i, 0, 0)),
            pl.BlockSpec((BN, D), lambda i: (i, 0)),
            pl.BlockSpec((R, D, D), lambda i: (0, 0, 0)),
            pl.BlockSpec((D, D), lambda i: (0, 0)),
        ],
        out_specs=pl.BlockSpec((BN, D), lambda i: (i, 0)),
        out_shape=jax.ShapeDtypeStruct((N, D), jnp.float32),
        compiler_params=pltpu.CompilerParams(
            dimension_semantics=("arbitrary",)
        ),
    )(s3, deg3, x, weight, self_weight)
    return out


# speedup vs baseline: 1.9272x; 1.1794x over previous
import functools

import jax
import jax.numpy as jnp
from jax import lax
from jax.experimental import pallas as pl
from jax.experimental.pallas import tpu as pltpu
from jax.experimental.pallas import tpu_sc as plsc

N = 10000
E = 320000
D = 128
R = 8

ADT = jnp.bfloat16
ROWS = R * N
TRASH = ROWS
SROWS = ROWS + 16
NC = 2
NS = 16
PASSES = 2
QROWS = 2 * N
QSTRIPE = QROWS // NS
IGN = -1

E2 = 2 * E
K = 256
TPE = 40960
NBLK = TPE // K
E2P = NS * TPE
NG = K // 16


_SC_PARAMS = pltpu.CompilerParams(
    needs_layout_passes=False, use_tc_tiling_on_sc=False
)
_MESH = dict(core_axis_name="c", subcore_axis_name="s",
             num_cores=NC, num_subcores=NS)

TPD = E2P // (NC * NS)
NBLKD = TPD // K
NBT = E2P // K


def _sc_degrees(epack, z2):
    @functools.partial(
        pl.kernel,
        mesh=plsc.VectorSubcoreMesh(**_MESH),
        compiler_params=_SC_PARAMS,
        out_type=jax.ShapeDtypeStruct((NC * NS, ROWS), jnp.float32),
        scratch_types=[
            pltpu.VMEM((2, 3, K), jnp.int32),
            pltpu.VMEM((SROWS,), jnp.float32),
            pltpu.SemaphoreType.DMA,
        ],
    )
    def deg_kernel(ep_h, z2_h, deg_out, est, hist, sem_e):
        cid = lax.axis_index("c")
        tid = lax.axis_index("s")
        wid = cid * NS + tid
        bbase = wid * NBLKD
        pltpu.sync_copy(z2_h, hist)
        ones = jnp.ones((16,), jnp.float32)
        pltpu.async_copy(ep_h.at[bbase], est.at[0], sem_e)

        @pl.loop(0, NBLKD)
        def _(b):
            q = lax.rem(b, 2)
            pltpu.make_async_copy(ep_h.at[0], est.at[0], sem_e).wait()
            nxt = jnp.minimum(b + 1, NBLKD - 1)
            pltpu.async_copy(ep_h.at[bbase + nxt], est.at[1 - q], sem_e)
            for j in range(NG):
                d = est[q, 1, pl.ds(j * 16, 16)]
                t = est[q, 2, pl.ds(j * 16, 16)]
                plsc.addupdate_scatter(hist, [t * N + d], ones)

        pltpu.make_async_copy(ep_h.at[0], est.at[0], sem_e).wait()
        pltpu.sync_copy(hist.at[pl.ds(0, ROWS)], deg_out.at[wid])

    return deg_kernel(epack, z2)


def _sc_segment_sums(xflat, epack, z1):
    @functools.partial(
        pl.kernel,
        mesh=plsc.VectorSubcoreMesh(**_MESH),
        compiler_params=_SC_PARAMS,
        out_type=jax.ShapeDtypeStruct((ROWS, D), ADT),
        scratch_types=[
            pltpu.VMEM((2, 3, K), jnp.int32),
            pltpu.VMEM((2, K), jnp.int32),
            pltpu.VMEM((2, K), jnp.int32),
            pltpu.VMEM((2, K, D), ADT),
            pltpu.VMEM_SHARED((QROWS, D), ADT),
            pltpu.SemaphoreType.DMA,
            pltpu.SemaphoreType.DMA,
            pltpu.SemaphoreType.DMA,
        ],
    )
    def sc_kernel(xf, ep_h, z1_h, s_out,
                  est, idx3, gidx3, rows2, s_sh, sem_e, sem_g, sem_s):
        cid = lax.axis_index("c")
        tid = lax.axis_index("s")
        bbase = tid * NBLK

        def start_edges(b, q):
            pltpu.async_copy(ep_h.at[bbase + b], est.at[q], sem_e)

        def wait_edges():
            pltpu.make_async_copy(ep_h.at[0], est.at[0], sem_e).wait()

        def compute_indices(q, qn):
            for j in range(NG):
                s = est[q, 0, pl.ds(j * 16, 16)]
                d = est[q, 1, pl.ds(j * 16, 16)]
                t = est[q, 2, pl.ds(j * 16, 16)]
                match = jnp.right_shift(t, 1) == qn
                idx3[q, pl.ds(j * 16, 16)] = jnp.where(
                    match, jnp.bitwise_and(t, 1) * N + d, IGN
                )
                gidx3[q, pl.ds(j * 16, 16)] = jnp.where(match, s, IGN)

        def gsrc(q):
            return xf.at[plsc.Indices(gidx3.at[q], ignored_value=IGN)]

        def sdst(q):
            return s_sh.at[plsc.Indices(idx3.at[q], ignored_value=IGN)]

        def fire_gathers(q):
            pltpu.async_copy(gsrc(q), rows2.at[q], sem_g)

        def wait_gathers(q):
            pltpu.make_async_copy(gsrc(q), rows2.at[q], sem_g).wait()

        def fire_scatters(q):
            pltpu.async_copy(rows2.at[q], sdst(q), sem_s, add=True)

        def wait_scatters(q):
            pltpu.make_async_copy(rows2.at[q], sdst(q), sem_s).wait()

        for p in range(PASSES):
            qn = cid * PASSES + p

            pltpu.sync_copy(z1_h, s_sh.at[pl.ds(tid * QSTRIPE, QSTRIPE)])
            plsc.subcore_barrier()

            start_edges(0, 0)
            wait_edges()
            compute_indices(0, qn)
            start_edges(1, 1)
            fire_gathers(0)
            wait_edges()
            compute_indices(1, qn)
            start_edges(2, 0)
            wait_gathers(0)
            fire_gathers(1)
            fire_scatters(0)

            @pl.loop(1, NBLK - 2)
            def _(b):
                q = lax.rem(b, 2)
                qo = 1 - q
                wait_scatters(qo)
                wait_edges()
                compute_indices(qo, qn)
                start_edges(b + 2, q)
                wait_gathers(q)
                fire_gathers(qo)
                fire_scatters(q)

            wait_scatters(1)
            wait_edges()
            compute_indices(1, qn)
            wait_gathers(0)
            fire_gathers(1)
            fire_scatters(0)
            wait_scatters(0)
            wait_gathers(1)
            fire_scatters(1)
            wait_scatters(1)

            plsc.subcore_barrier()
            pltpu.sync_copy(
                s_sh.at[pl.ds(tid * QSTRIPE, QSTRIPE)],
                s_out.at[pl.ds(qn * QROWS + tid * QSTRIPE, QSTRIPE)],
            )

    return sc_kernel(xflat, epack, z1)


BN = 1000


def _tc_body(s_ref, deg_ref, x_ref, w_ref, sw_ref, o_ref):
    deg = jnp.sum(deg_ref[...], axis=2)
    dinv = jnp.where(deg > 0.0, 1.0 / jnp.maximum(deg, 1.0), 0.0)
    acc = jnp.dot(x_ref[...], sw_ref[...], preferred_element_type=jnp.float32)
    for r in range(R):
        acc = acc + jnp.dot(
            s_ref[r].astype(jnp.float32) * dinv[:, r : r + 1], w_ref[r],
            preferred_element_type=jnp.float32,
        )
    o_ref[...] = acc


def kernel(x, edge_index, edge_type, weight, self_weight):
    src0 = edge_index[0]
    dst0 = edge_index[1]
    npad = E2P - E2
    srcp = jnp.concatenate([src0, dst0, jnp.full((npad,), N, jnp.int32)])
    dstp = jnp.concatenate([dst0, src0, jnp.zeros((npad,), jnp.int32)])
    etp = jnp.concatenate([edge_type, edge_type, jnp.full((npad,), R, jnp.int32)])
    epack = (
        jnp.stack([srcp, dstp, etp]).reshape(3, NBT, K).transpose(1, 0, 2)
    )
    xflat = x.astype(ADT)
    z1 = jnp.zeros((QSTRIPE, D), ADT)
    z2 = jnp.zeros((SROWS,), jnp.float32)

    deg32 = _sc_degrees(epack, z2)
    s_rows = _sc_segment_sums(xflat, epack, z1)
    s3 = s_rows.reshape(R, N, D)
    deg3 = deg32.reshape(NC * NS, R, N).transpose(2, 1, 0)

    out = pl.pallas_call(
        _tc_body,
        grid=(N // BN,),
        in_specs=[
            pl.BlockSpec((R, BN, D), lambda i: (0, i, 0)),
            pl.BlockSpec((BN, R, NC * NS), lambda i: (i, 0, 0)),
            pl.BlockSpec((BN, D), lambda i: (i, 0)),
            pl.BlockSpec((R, D, D), lambda i: (0, 0, 0)),
            pl.BlockSpec((D, D), lambda i: (0, 0)),
        ],
        out_specs=pl.BlockSpec((BN, D), lambda i: (i, 0)),
        out_shape=jax.ShapeDtypeStruct((N, D), jnp.float32),
        compiler_params=pltpu.CompilerParams(
            dimension_semantics=("arbitrary",)
        ),
    )(s3, deg3, x, weight, self_weight)
    return out
